# baseline (speedup 1.0000x reference)
import functools

import jax
import jax.numpy as jnp
from jax import lax
from jax.experimental import pallas as pl
from jax.experimental.pallas import tpu as pltpu
from jax.experimental.pallas import tpu_sc as plsc

N = 10000
E = 320000
NNZ_X = 500000
IN_DIM = 128
OUT_DIM = 128

NC = 2
NS = 16
L = 16

DN_CHUNK = 128
DN_SUPER = 2048
DN_SUPERS_PER_TILE = 8
DN_PER_TILE = DN_SUPERS_PER_TILE * DN_SUPER
DN_PADDED = NC * NS * DN_PER_TILE
ACC_WORDS = N * IN_DIM
DN_ZSTRIPE = ACC_WORDS // NS
DN_STAGE = 8000

EG_CHUNK = 128
EG_GRP = 16
EG_GROUPS = 5
EG_CHUNKS_PER_TILE = EG_GRP * EG_GROUPS
EG_PER_TILE = EG_CHUNKS_PER_TILE * EG_CHUNK
EG_GRP_E = EG_GRP * EG_CHUNK
EG_VSTRIDE = EG_GRP_E + L
EG_PADDED = NC * NS * EG_PER_TILE
EG_ROWBLK = 16
EG_NBLK = N // EG_ROWBLK


def _densify_body(rows_hbm, cols_hbm, vals_hbm, out_hbm,
                  r_buf, c_buf, v_buf, idx2d, stage, sem, acc):
    cid = lax.axis_index("c")
    sid = lax.axis_index("s")

    def _z(i, _):
        stage[pl.ds(i * L, L)] = jnp.zeros((L,), jnp.float32)
        return 0
    lax.fori_loop(0, DN_STAGE // L, _z, 0)
    for k in range(DN_ZSTRIPE // DN_STAGE):
        pltpu.sync_copy(stage, acc.at[pl.ds(sid * DN_ZSTRIPE + k * DN_STAGE,
                                            DN_STAGE)])
    plsc.subcore_barrier()

    t_base = (cid * NS + sid) * DN_PER_TILE

    def _super(s, _):
        off = t_base + s * DN_SUPER
        pltpu.sync_copy(rows_hbm.at[pl.ds(off, DN_SUPER)], r_buf)
        pltpu.sync_copy(cols_hbm.at[pl.ds(off, DN_SUPER)], c_buf)
        pltpu.sync_copy(vals_hbm.at[pl.ds(off, DN_SUPER)], v_buf)

        def _cmp(j, _):
            rv = r_buf[pl.ds(j * L, L)]
            cv = c_buf[pl.ds(j * L, L)]
            flat = rv * IN_DIM + cv
            idx2d[j // (DN_CHUNK // L), pl.ds((j % (DN_CHUNK // L)) * L, L)] = flat
            return 0
        for j in range(DN_SUPER // L):
            _cmp(j, 0)

        descs = [pltpu.async_copy(v_buf.at[pl.ds(k * DN_CHUNK, DN_CHUNK)],
                                  acc.at[idx2d.at[k]], sem, add=True)
                 for k in range(DN_SUPER // DN_CHUNK)]
        for d in descs:
            d.wait()
        return 0
    lax.fori_loop(0, DN_SUPERS_PER_TILE, _super, 0)
    plsc.subcore_barrier()

    for k in range(DN_ZSTRIPE // DN_STAGE):
        off = sid * DN_ZSTRIPE + k * DN_STAGE
        pltpu.sync_copy(acc.at[pl.ds(off, DN_STAGE)], stage)
        pltpu.sync_copy(stage, out_hbm.at[pl.ds(cid * ACC_WORDS + off, DN_STAGE)])


_densify = functools.partial(
    pl.kernel,
    out_type=jax.ShapeDtypeStruct((NC * N * IN_DIM,), jnp.float32),
    mesh=plsc.VectorSubcoreMesh(core_axis_name="c", subcore_axis_name="s"),
    scratch_types=[
        pltpu.VMEM((DN_SUPER,), jnp.int32),
        pltpu.VMEM((DN_SUPER,), jnp.int32),
        pltpu.VMEM((DN_SUPER,), jnp.float32),
        pltpu.VMEM((DN_SUPER // DN_CHUNK, DN_CHUNK), jnp.int32),
        pltpu.VMEM((DN_STAGE,), jnp.float32),
        pltpu.SemaphoreType.DMA,
        pltpu.VMEM_SHARED((ACC_WORDS,), jnp.float32),
    ],
)(_densify_body)


def _sum2_body(p_ref, o_ref):
    o_ref[...] = p_ref[0] + p_ref[1]


def _sum2(parts):
    return pl.pallas_call(
        _sum2_body,
        grid=(N // _BM,),
        in_specs=[pl.BlockSpec((NC, _BM * IN_DIM), lambda i: (0, i))],
        out_specs=pl.BlockSpec((_BM * IN_DIM,), lambda i: (i,)),
        out_shape=jax.ShapeDtypeStruct((N * IN_DIM,), jnp.float32),
    )(parts.reshape(NC, N * IN_DIM))


def _edge_body(src_hbm, dst_hbm, vals_hbm, xd_hbm, out_hbm,
               s2d, d2d, v_vmem, rows_buf, zrows, sem0, sem1, acc):
    cid = lax.axis_index("c")
    sid = lax.axis_index("s")
    wid = sid * NC + cid

    def _z(r, _):
        for j in range(IN_DIM // L):
            zrows[r, pl.ds(j * L, L)] = jnp.zeros((L,), jnp.float32)
        return 0
    lax.fori_loop(0, EG_ROWBLK, _z, 0)

    def _zero_blk(k, _):
        blk = k * NS + sid

        @pl.when(blk < EG_NBLK)
        def _():
            pltpu.sync_copy(zrows, acc.at[pl.ds(blk * EG_ROWBLK, EG_ROWBLK)])
        return 0
    lax.fori_loop(0, (EG_NBLK + NS - 1) // NS, _zero_blk, 0)
    plsc.subcore_barrier()

    e_base = wid * EG_PER_TILE
    sems = (sem0, sem1)

    def _loads(c, bi):
        off = e_base + c * EG_CHUNK
        pltpu.sync_copy(src_hbm.at[pl.ds(off, EG_CHUNK)], s2d.at[bi])
        pltpu.sync_copy(dst_hbm.at[pl.ds(off, EG_CHUNK)], d2d.at[bi])
        pltpu.sync_copy(vals_hbm.at[pl.ds(off, EG_CHUNK)],
                        v_vmem.at[pl.ds(bi * EG_CHUNK, EG_CHUNK)])

    def _issue_gather(bi):
        pltpu.async_copy(xd_hbm.at[s2d.at[bi]], rows_buf.at[bi], sems[bi])

    def _consume(bi):
        pltpu.make_async_copy(xd_hbm.at[s2d.at[bi]], rows_buf.at[bi],
                              sems[bi]).wait()

        def _scale(i, _):
            val = jnp.full((L,), v_vmem[pl.ds(bi * EG_CHUNK + i, L)][0],
                           jnp.float32)
            for j in range(IN_DIM // L):
                rows_buf[bi, i, pl.ds(j * L, L)] = (
                    rows_buf[bi, i, pl.ds(j * L, L)] * val)
            return 0
        lax.fori_loop(0, EG_CHUNK, _scale, 0)
        pltpu.sync_copy(rows_buf.at[bi], acc.at[d2d.at[bi]], add=True)

    _loads(0, 0)
    _issue_gather(0)
    _loads(1, 1)
    _issue_gather(1)

    def _pair(k, _):
        c0 = k * 2
        _consume(0)

        @pl.when(k < EG_CHUNKS_PER_TILE // 2 - 1)
        def _():
            _loads(c0 + 2, 0)
            _issue_gather(0)
        _consume(1)

        @pl.when(k < EG_CHUNKS_PER_TILE // 2 - 1)
        def _():
            _loads(c0 + 3, 1)
            _issue_gather(1)
        return 0
    lax.fori_loop(0, EG_CHUNKS_PER_TILE // 2, _pair, 0)
    plsc.subcore_barrier()

    def _wb_blk(k, _):
        blk = k * NS + sid

        @pl.when(blk < EG_NBLK)
        def _():
            r0 = blk * EG_ROWBLK
            pltpu.sync_copy(acc.at[pl.ds(r0, EG_ROWBLK)], zrows)
            pltpu.sync_copy(zrows, out_hbm.at[cid, pl.ds(r0, EG_ROWBLK)])
        return 0
    lax.fori_loop(0, (EG_NBLK + NS - 1) // NS, _wb_blk, 0)


_edge_agg = functools.partial(
    pl.kernel,
    out_type=jax.ShapeDtypeStruct((NC, N, OUT_DIM), jnp.float32),
    mesh=plsc.VectorSubcoreMesh(core_axis_name="c", subcore_axis_name="s"),
    scratch_types=[
        pltpu.VMEM((2, EG_CHUNK), jnp.int32),
        pltpu.VMEM((2, EG_CHUNK), jnp.int32),
        pltpu.VMEM((2 * EG_CHUNK + L,), jnp.float32),
        pltpu.VMEM((2, EG_CHUNK, IN_DIM), jnp.float32),
        pltpu.VMEM((EG_ROWBLK, IN_DIM), jnp.float32),
        pltpu.SemaphoreType.DMA,
        pltpu.SemaphoreType.DMA,
        pltpu.VMEM_SHARED((N, IN_DIM), jnp.float32),
    ],
)(_edge_body)


def _matmul_body(p_ref, w_ref, o_ref):
    x = p_ref[0] + p_ref[1]
    y = jnp.dot(x, w_ref[...], preferred_element_type=jnp.float32)
    o_ref[...] = jnp.maximum(y, 0.0)


_BM = 1000


def _matmul_relu(parts, W):
    return pl.pallas_call(
        _matmul_body,
        grid=(N // _BM,),
        in_specs=[
            pl.BlockSpec((NC, _BM, IN_DIM), lambda i: (0, i, 0)),
            pl.BlockSpec((IN_DIM, OUT_DIM), lambda i: (0, 0)),
        ],
        out_specs=pl.BlockSpec((_BM, OUT_DIM), lambda i: (i, 0)),
        out_shape=jax.ShapeDtypeStruct((N, OUT_DIM), jnp.float32),
    )(parts, W)


def kernel(x_rows, x_cols, x_vals, edge_index, adj_vals, W):
    dpad = DN_PADDED - NNZ_X
    xr = jnp.pad(x_rows.astype(jnp.int32), (0, dpad))
    xc = jnp.pad(x_cols.astype(jnp.int32), (0, dpad))
    xv = jnp.pad(x_vals, (0, dpad))

    epad = EG_PADDED - E
    src = jnp.pad(edge_index[1].astype(jnp.int32), (0, epad))
    dst = jnp.pad(edge_index[0].astype(jnp.int32), (0, epad))
    av = jnp.pad(adj_vals, (0, epad))

    xd = _sum2(_densify(xr, xc, xv)).reshape(N, IN_DIM)
    parts = _edge_agg(src, dst, av, xd)
    return _matmul_relu(parts, W)


# speedup vs baseline: 1.0112x; 1.0112x over previous
import functools

import jax
import jax.numpy as jnp
from jax import lax
from jax.experimental import pallas as pl
from jax.experimental.pallas import tpu as pltpu
from jax.experimental.pallas import tpu_sc as plsc

N = 10000
E = 320000
NNZ_X = 500000
IN_DIM = 128
OUT_DIM = 128

NC = 2
NS = 16
L = 16

DN_CHUNK = 128
DN_SUPER = 2048
DN_SUPERS_PER_TILE = 8
DN_PER_TILE = DN_SUPERS_PER_TILE * DN_SUPER
DN_PADDED = NC * NS * DN_PER_TILE
ACC_WORDS = N * IN_DIM
DN_ZSTRIPE = ACC_WORDS // NS
DN_STAGE = 8000

EG_CHUNK = 128
EG_GRP = 16
EG_GROUPS = 5
EG_CHUNKS_PER_TILE = EG_GRP * EG_GROUPS
EG_PER_TILE = EG_CHUNKS_PER_TILE * EG_CHUNK
EG_GRP_E = EG_GRP * EG_CHUNK
EG_VSTRIDE = EG_GRP_E + L
EG_PADDED = NC * NS * EG_PER_TILE
EG_ROWBLK = 64
EG_NBLK = N // EG_ROWBLK
EG_REM = N - EG_NBLK * EG_ROWBLK


def _densify_body(rows_hbm, cols_hbm, vals_hbm, out_hbm,
                  r_buf, c_buf, v_buf, idx2d, stage, sem, acc):
    cid = lax.axis_index("c")
    sid = lax.axis_index("s")

    def _z(i, _):
        stage[pl.ds(i * L, L)] = jnp.zeros((L,), jnp.float32)
        return 0
    lax.fori_loop(0, DN_STAGE // L, _z, 0)
    for k in range(DN_ZSTRIPE // DN_STAGE):
        pltpu.sync_copy(stage, acc.at[pl.ds(sid * DN_ZSTRIPE + k * DN_STAGE,
                                            DN_STAGE)])
    plsc.subcore_barrier()

    t_base = (cid * NS + sid) * DN_PER_TILE

    def _super(s, _):
        off = t_base + s * DN_SUPER
        pltpu.sync_copy(rows_hbm.at[pl.ds(off, DN_SUPER)], r_buf)
        pltpu.sync_copy(cols_hbm.at[pl.ds(off, DN_SUPER)], c_buf)
        pltpu.sync_copy(vals_hbm.at[pl.ds(off, DN_SUPER)], v_buf)

        def _cmp(j, _):
            rv = r_buf[pl.ds(j * L, L)]
            cv = c_buf[pl.ds(j * L, L)]
            flat = rv * IN_DIM + cv
            idx2d[j // (DN_CHUNK // L), pl.ds((j % (DN_CHUNK // L)) * L, L)] = flat
            return 0
        for j in range(DN_SUPER // L):
            _cmp(j, 0)

        descs = [pltpu.async_copy(v_buf.at[pl.ds(k * DN_CHUNK, DN_CHUNK)],
                                  acc.at[idx2d.at[k]], sem, add=True)
                 for k in range(DN_SUPER // DN_CHUNK)]
        for d in descs:
            d.wait()
        return 0
    lax.fori_loop(0, DN_SUPERS_PER_TILE, _super, 0)
    plsc.subcore_barrier()

    for k in range(DN_ZSTRIPE // DN_STAGE):
        off = sid * DN_ZSTRIPE + k * DN_STAGE
        pltpu.sync_copy(acc.at[pl.ds(off, DN_STAGE)], stage)
        pltpu.sync_copy(stage, out_hbm.at[pl.ds(cid * ACC_WORDS + off, DN_STAGE)])


_densify = functools.partial(
    pl.kernel,
    out_type=jax.ShapeDtypeStruct((NC * N * IN_DIM,), jnp.float32),
    mesh=plsc.VectorSubcoreMesh(core_axis_name="c", subcore_axis_name="s"),
    scratch_types=[
        pltpu.VMEM((DN_SUPER,), jnp.int32),
        pltpu.VMEM((DN_SUPER,), jnp.int32),
        pltpu.VMEM((DN_SUPER,), jnp.float32),
        pltpu.VMEM((DN_SUPER // DN_CHUNK, DN_CHUNK), jnp.int32),
        pltpu.VMEM((DN_STAGE,), jnp.float32),
        pltpu.SemaphoreType.DMA,
        pltpu.VMEM_SHARED((ACC_WORDS,), jnp.float32),
    ],
)(_densify_body)


def _sum2_body(p_ref, o_ref):
    o_ref[...] = p_ref[0] + p_ref[1]


def _sum2(parts):
    return pl.pallas_call(
        _sum2_body,
        grid=(N // _BM,),
        in_specs=[pl.BlockSpec((NC, _BM * IN_DIM), lambda i: (0, i))],
        out_specs=pl.BlockSpec((_BM * IN_DIM,), lambda i: (i,)),
        out_shape=jax.ShapeDtypeStruct((N * IN_DIM,), jnp.float32),
    )(parts.reshape(NC, N * IN_DIM))


def _edge_body(src_hbm, dst_hbm, vals_hbm, xd_hbm, out_hbm,
               s2d, d2d, v_vmem, rows_buf, zrows, sem0, sem1, acc):
    cid = lax.axis_index("c")
    sid = lax.axis_index("s")
    wid = sid * NC + cid

    def _z(r, _):
        for j in range(IN_DIM // L):
            zrows[r, pl.ds(j * L, L)] = jnp.zeros((L,), jnp.float32)
        return 0
    lax.fori_loop(0, EG_ROWBLK, _z, 0)

    def _zero_blk(k, _):
        blk = k * NS + sid

        @pl.when(blk < EG_NBLK)
        def _():
            pltpu.sync_copy(zrows, acc.at[pl.ds(blk * EG_ROWBLK, EG_ROWBLK)])
        return 0
    lax.fori_loop(0, (EG_NBLK + NS - 1) // NS, _zero_blk, 0)

    @pl.when(sid == 0)
    def _():
        pltpu.sync_copy(zrows.at[pl.ds(0, EG_REM)],
                        acc.at[pl.ds(EG_NBLK * EG_ROWBLK, EG_REM)])
    plsc.subcore_barrier()

    e_base = wid * EG_PER_TILE
    sems = (sem0, sem1)

    def _loads(c, bi):
        off = e_base + c * EG_CHUNK
        pltpu.sync_copy(src_hbm.at[pl.ds(off, EG_CHUNK)], s2d.at[bi])
        pltpu.sync_copy(dst_hbm.at[pl.ds(off, EG_CHUNK)], d2d.at[bi])
        pltpu.sync_copy(vals_hbm.at[pl.ds(off, EG_CHUNK)],
                        v_vmem.at[pl.ds(bi * EG_CHUNK, EG_CHUNK)])

    def _issue_gather(bi):
        pltpu.async_copy(xd_hbm.at[s2d.at[bi]], rows_buf.at[bi], sems[bi])

    def _consume(bi):
        pltpu.make_async_copy(xd_hbm.at[s2d.at[bi]], rows_buf.at[bi],
                              sems[bi]).wait()

        def _scale(i, _):
            val = jnp.full((L,), v_vmem[pl.ds(bi * EG_CHUNK + i, L)][0],
                           jnp.float32)
            for j in range(IN_DIM // L):
                rows_buf[bi, i, pl.ds(j * L, L)] = (
                    rows_buf[bi, i, pl.ds(j * L, L)] * val)
            return 0
        lax.fori_loop(0, EG_CHUNK, _scale, 0)
        pltpu.sync_copy(rows_buf.at[bi], acc.at[d2d.at[bi]], add=True)

    _loads(0, 0)
    _issue_gather(0)
    _loads(1, 1)
    _issue_gather(1)

    def _pair(k, _):
        c0 = k * 2
        _consume(0)

        @pl.when(k < EG_CHUNKS_PER_TILE // 2 - 1)
        def _():
            _loads(c0 + 2, 0)
            _issue_gather(0)
        _consume(1)

        @pl.when(k < EG_CHUNKS_PER_TILE // 2 - 1)
        def _():
            _loads(c0 + 3, 1)
            _issue_gather(1)
        return 0
    lax.fori_loop(0, EG_CHUNKS_PER_TILE // 2, _pair, 0)
    plsc.subcore_barrier()

    def _wb_blk(k, _):
        blk = k * NS + sid

        @pl.when(blk < EG_NBLK)
        def _():
            r0 = blk * EG_ROWBLK
            pltpu.sync_copy(acc.at[pl.ds(r0, EG_ROWBLK)], zrows)
            pltpu.sync_copy(zrows, out_hbm.at[cid, pl.ds(r0, EG_ROWBLK)])
        return 0
    lax.fori_loop(0, (EG_NBLK + NS - 1) // NS, _wb_blk, 0)

    @pl.when(sid == 0)
    def _():
        r0 = EG_NBLK * EG_ROWBLK
        pltpu.sync_copy(acc.at[pl.ds(r0, EG_REM)], zrows.at[pl.ds(0, EG_REM)])
        pltpu.sync_copy(zrows.at[pl.ds(0, EG_REM)],
                        out_hbm.at[cid, pl.ds(r0, EG_REM)])


_edge_agg = functools.partial(
    pl.kernel,
    out_type=jax.ShapeDtypeStruct((NC, N, OUT_DIM), jnp.float32),
    mesh=plsc.VectorSubcoreMesh(core_axis_name="c", subcore_axis_name="s"),
    scratch_types=[
        pltpu.VMEM((2, EG_CHUNK), jnp.int32),
        pltpu.VMEM((2, EG_CHUNK), jnp.int32),
        pltpu.VMEM((2 * EG_CHUNK + L,), jnp.float32),
        pltpu.VMEM((2, EG_CHUNK, IN_DIM), jnp.float32),
        pltpu.VMEM((EG_ROWBLK, IN_DIM), jnp.float32),
        pltpu.SemaphoreType.DMA,
        pltpu.SemaphoreType.DMA,
        pltpu.VMEM_SHARED((N, IN_DIM), jnp.float32),
    ],
)(_edge_body)


def _matmul_body(p_ref, w_ref, o_ref):
    x = p_ref[0] + p_ref[1]
    y = jnp.dot(x, w_ref[...], preferred_element_type=jnp.float32)
    o_ref[...] = jnp.maximum(y, 0.0)


_BM = 1000


def _matmul_relu(parts, W):
    return pl.pallas_call(
        _matmul_body,
        grid=(N // _BM,),
        in_specs=[
            pl.BlockSpec((NC, _BM, IN_DIM), lambda i: (0, i, 0)),
            pl.BlockSpec((IN_DIM, OUT_DIM), lambda i: (0, 0)),
        ],
        out_specs=pl.BlockSpec((_BM, OUT_DIM), lambda i: (i, 0)),
        out_shape=jax.ShapeDtypeStruct((N, OUT_DIM), jnp.float32),
    )(parts, W)


def kernel(x_rows, x_cols, x_vals, edge_index, adj_vals, W):
    dpad = DN_PADDED - NNZ_X
    xr = jnp.pad(x_rows.astype(jnp.int32), (0, dpad))
    xc = jnp.pad(x_cols.astype(jnp.int32), (0, dpad))
    xv = jnp.pad(x_vals, (0, dpad))

    epad = EG_PADDED - E
    src = jnp.pad(edge_index[1].astype(jnp.int32), (0, epad))
    dst = jnp.pad(edge_index[0].astype(jnp.int32), (0, epad))
    av = jnp.pad(adj_vals, (0, epad))

    xd = _sum2(_densify(xr, xc, xv)).reshape(N, IN_DIM)
    parts = _edge_agg(src, dst, av, xd)
    return _matmul_relu(parts, W)


# speedup vs baseline: 1.6038x; 1.5861x over previous
import functools

import jax
import jax.numpy as jnp
from jax import lax
from jax.experimental import pallas as pl
from jax.experimental.pallas import tpu as pltpu
from jax.experimental.pallas import tpu_sc as plsc

N = 10000
E = 320000
NNZ_X = 500000
IN_DIM = 128
OUT_DIM = 128

NC = 2
NS = 16
L = 16

DN_CHUNK = 128
DN_SUPER = 2048
DN_SUPERS_PER_TILE = 8
DN_PER_TILE = DN_SUPERS_PER_TILE * DN_SUPER
DN_PADDED = NC * NS * DN_PER_TILE
ACC_WORDS = N * IN_DIM
DN_ZSTRIPE = ACC_WORDS // NS
DN_STAGE = 8000

EG_CHUNK = 128
EG_GRP = 16
EG_GROUPS = 5
EG_CHUNKS_PER_TILE = EG_GRP * EG_GROUPS
EG_PER_TILE = EG_CHUNKS_PER_TILE * EG_CHUNK
EG_GRP_E = EG_GRP * EG_CHUNK
EG_VSTRIDE = EG_GRP_E + L
EG_PADDED = NC * NS * EG_PER_TILE
EG_ROWBLK = 64
EG_NBLK = N // EG_ROWBLK
EG_REM = N - EG_NBLK * EG_ROWBLK


def _densify_body(rows_hbm, cols_hbm, vals_hbm, out_hbm,
                  r_buf, c_buf, v_buf, idx2d, stage, sem, acc):
    cid = lax.axis_index("c")
    sid = lax.axis_index("s")

    def _z(i, _):
        stage[pl.ds(i * L, L)] = jnp.zeros((L,), jnp.float32)
        return 0
    lax.fori_loop(0, DN_STAGE // L, _z, 0)
    for k in range(DN_ZSTRIPE // DN_STAGE):
        pltpu.sync_copy(stage, acc.at[pl.ds(sid * DN_ZSTRIPE + k * DN_STAGE,
                                            DN_STAGE)])
    plsc.subcore_barrier()

    t_base = (cid * NS + sid) * DN_PER_TILE

    def _super(s, _):
        off = t_base + s * DN_SUPER
        pltpu.sync_copy(rows_hbm.at[pl.ds(off, DN_SUPER)], r_buf)
        pltpu.sync_copy(cols_hbm.at[pl.ds(off, DN_SUPER)], c_buf)
        pltpu.sync_copy(vals_hbm.at[pl.ds(off, DN_SUPER)], v_buf)

        def _cmp(j, _):
            rv = r_buf[pl.ds(j * L, L)]
            cv = c_buf[pl.ds(j * L, L)]
            flat = rv * IN_DIM + cv
            idx2d[j // (DN_CHUNK // L), pl.ds((j % (DN_CHUNK // L)) * L, L)] = flat
            return 0
        for j in range(DN_SUPER // L):
            _cmp(j, 0)

        descs = [pltpu.async_copy(v_buf.at[pl.ds(k * DN_CHUNK, DN_CHUNK)],
                                  acc.at[idx2d.at[k]], sem, add=True)
                 for k in range(DN_SUPER // DN_CHUNK)]
        for d in descs:
            d.wait()
        return 0
    lax.fori_loop(0, DN_SUPERS_PER_TILE, _super, 0)
    plsc.subcore_barrier()

    for k in range(DN_ZSTRIPE // DN_STAGE):
        off = sid * DN_ZSTRIPE + k * DN_STAGE
        pltpu.sync_copy(acc.at[pl.ds(off, DN_STAGE)], stage)
        pltpu.sync_copy(stage, out_hbm.at[pl.ds(cid * ACC_WORDS + off, DN_STAGE)])


_densify = functools.partial(
    pl.kernel,
    out_type=jax.ShapeDtypeStruct((NC * N * IN_DIM,), jnp.float32),
    mesh=plsc.VectorSubcoreMesh(core_axis_name="c", subcore_axis_name="s"),
    scratch_types=[
        pltpu.VMEM((DN_SUPER,), jnp.int32),
        pltpu.VMEM((DN_SUPER,), jnp.int32),
        pltpu.VMEM((DN_SUPER,), jnp.float32),
        pltpu.VMEM((DN_SUPER // DN_CHUNK, DN_CHUNK), jnp.int32),
        pltpu.VMEM((DN_STAGE,), jnp.float32),
        pltpu.SemaphoreType.DMA,
        pltpu.VMEM_SHARED((ACC_WORDS,), jnp.float32),
    ],
)(_densify_body)


def _sum2_body(p_ref, o_ref):
    o_ref[...] = p_ref[0] + p_ref[1]


def _sum2(parts):
    return pl.pallas_call(
        _sum2_body,
        grid=(N // _BM,),
        in_specs=[pl.BlockSpec((NC, _BM * IN_DIM), lambda i: (0, i))],
        out_specs=pl.BlockSpec((_BM * IN_DIM,), lambda i: (i,)),
        out_shape=jax.ShapeDtypeStruct((N * IN_DIM,), jnp.float32),
    )(parts.reshape(NC, N * IN_DIM))


def _edge_body(src_hbm, dst_hbm, vals_hbm, xd_hbm, out_hbm,
               s2d, d2d, v_vmem, rows_buf, zrows, sem0, sem1, acc):
    cid = lax.axis_index("c")
    sid = lax.axis_index("s")
    wid = sid * NC + cid

    def _z(r, _):
        for j in range(IN_DIM // L):
            zrows[r, pl.ds(j * L, L)] = jnp.zeros((L,), jnp.float32)
        return 0
    lax.fori_loop(0, EG_ROWBLK, _z, 0)

    def _zero_blk(k, _):
        blk = k * NS + sid

        @pl.when(blk < EG_NBLK)
        def _():
            pltpu.sync_copy(zrows, acc.at[pl.ds(blk * EG_ROWBLK, EG_ROWBLK)])
        return 0
    lax.fori_loop(0, (EG_NBLK + NS - 1) // NS, _zero_blk, 0)

    @pl.when(sid == 0)
    def _():
        pltpu.sync_copy(zrows.at[pl.ds(0, EG_REM)],
                        acc.at[pl.ds(EG_NBLK * EG_ROWBLK, EG_REM)])
    plsc.subcore_barrier()

    e_base = wid * EG_PER_TILE
    sems = (sem0, sem1)

    def _loads(c, bi):
        off = e_base + c * EG_CHUNK
        pltpu.sync_copy(src_hbm.at[pl.ds(off, EG_CHUNK)], s2d.at[bi])
        pltpu.sync_copy(dst_hbm.at[pl.ds(off, EG_CHUNK)], d2d.at[bi])
        pltpu.sync_copy(vals_hbm.at[pl.ds(off, EG_CHUNK)],
                        v_vmem.at[pl.ds(bi * EG_CHUNK, EG_CHUNK)])

    def _issue_gather(bi):
        pltpu.async_copy(xd_hbm.at[s2d.at[bi]], rows_buf.at[bi], sems[bi])

    def _consume(bi):
        pltpu.make_async_copy(xd_hbm.at[s2d.at[bi]], rows_buf.at[bi],
                              sems[bi]).wait()

        def _scale(i, _):
            val = jnp.full((L,), v_vmem[pl.ds(bi * EG_CHUNK + i, L)][0],
                           jnp.float32)
            for j in range(IN_DIM // L):
                rows_buf[bi, i, pl.ds(j * L, L)] = (
                    rows_buf[bi, i, pl.ds(j * L, L)] * val)
            return 0
        lax.fori_loop(0, EG_CHUNK, _scale, 0)
        pltpu.sync_copy(rows_buf.at[bi], acc.at[d2d.at[bi]], add=True)

    _loads(0, 0)
    _issue_gather(0)
    _loads(1, 1)
    _issue_gather(1)

    def _pair(k, _):
        c0 = k * 2
        _consume(0)

        @pl.when(k < EG_CHUNKS_PER_TILE // 2 - 1)
        def _():
            _loads(c0 + 2, 0)
            _issue_gather(0)
        _consume(1)

        @pl.when(k < EG_CHUNKS_PER_TILE // 2 - 1)
        def _():
            _loads(c0 + 3, 1)
            _issue_gather(1)
        return 0
    lax.fori_loop(0, EG_CHUNKS_PER_TILE // 2, _pair, 0)
    plsc.subcore_barrier()

    def _wb_blk(k, _):
        blk = k * NS + sid

        @pl.when(blk < EG_NBLK)
        def _():
            r0 = blk * EG_ROWBLK
            pltpu.sync_copy(acc.at[pl.ds(r0, EG_ROWBLK)], zrows)
            pltpu.sync_copy(zrows, out_hbm.at[cid, pl.ds(r0, EG_ROWBLK)])
        return 0
    lax.fori_loop(0, (EG_NBLK + NS - 1) // NS, _wb_blk, 0)

    @pl.when(sid == 0)
    def _():
        r0 = EG_NBLK * EG_ROWBLK
        pltpu.sync_copy(acc.at[pl.ds(r0, EG_REM)], zrows.at[pl.ds(0, EG_REM)])
        pltpu.sync_copy(zrows.at[pl.ds(0, EG_REM)],
                        out_hbm.at[cid, pl.ds(r0, EG_REM)])


_edge_agg = functools.partial(
    pl.kernel,
    out_type=jax.ShapeDtypeStruct((NC, N, OUT_DIM), jnp.float32),
    mesh=plsc.VectorSubcoreMesh(core_axis_name="c", subcore_axis_name="s"),
    scratch_types=[
        pltpu.VMEM((2, EG_CHUNK), jnp.int32),
        pltpu.VMEM((2, EG_CHUNK), jnp.int32),
        pltpu.VMEM((2 * EG_CHUNK + L,), jnp.float32),
        pltpu.VMEM((2, EG_CHUNK, IN_DIM), jnp.float32),
        pltpu.VMEM((EG_ROWBLK, IN_DIM), jnp.float32),
        pltpu.SemaphoreType.DMA,
        pltpu.SemaphoreType.DMA,
        pltpu.VMEM_SHARED((N, IN_DIM), jnp.float32),
    ],
)(_edge_body)


def _matmul_body(p_ref, w_ref, o_ref):
    x = p_ref[0] + p_ref[1]
    y = jnp.dot(x, w_ref[...], preferred_element_type=jnp.float32)
    o_ref[...] = jnp.maximum(y, 0.0)


_BM = 1000


def _matmul_relu(parts, W):
    return pl.pallas_call(
        _matmul_body,
        grid=(N // _BM,),
        in_specs=[
            pl.BlockSpec((NC, _BM, IN_DIM), lambda i: (0, i, 0)),
            pl.BlockSpec((IN_DIM, OUT_DIM), lambda i: (0, 0)),
        ],
        out_specs=pl.BlockSpec((_BM, OUT_DIM), lambda i: (i, 0)),
        out_shape=jax.ShapeDtypeStruct((N, OUT_DIM), jnp.float32),
    )(parts, W)


def kernel(x_rows, x_cols, x_vals, edge_index, adj_vals, W):
    dpad = DN_PADDED - NNZ_X
    dfill = jnp.arange(dpad, dtype=jnp.int32)
    xr = jnp.concatenate([x_rows.astype(jnp.int32), dfill % N])
    xc = jnp.concatenate([x_cols.astype(jnp.int32), dfill % IN_DIM])
    xv = jnp.pad(x_vals, (0, dpad))

    epad = EG_PADDED - E
    efill = jnp.arange(epad, dtype=jnp.int32) % N
    src = jnp.concatenate([edge_index[1].astype(jnp.int32), efill])
    dst = jnp.concatenate([edge_index[0].astype(jnp.int32), efill])
    av = jnp.pad(adj_vals, (0, epad))

    xd = _sum2(_densify(xr, xc, xv)).reshape(N, IN_DIM)
    parts = _edge_agg(src, dst, av, xd)
    return _matmul_relu(parts, W)


# speedup vs baseline: 1.7931x; 1.1180x over previous
import functools

import jax
import jax.numpy as jnp
from jax import lax
from jax.experimental import pallas as pl
from jax.experimental.pallas import tpu as pltpu
from jax.experimental.pallas import tpu_sc as plsc

N = 10000
E = 320000
NNZ_X = 500000
IN_DIM = 128
OUT_DIM = 128

NC = 2
NS = 16
L = 16

DN_CHUNK = 128
DN_SUPER = 2048
DN_SUPERS_PER_TILE = 8
DN_PER_TILE = DN_SUPERS_PER_TILE * DN_SUPER
DN_PADDED = NC * NS * DN_PER_TILE
ACC_WORDS = N * IN_DIM
DN_ZSTRIPE = ACC_WORDS // NS
DN_STAGE = 8000

EG_CHUNK = 120
EG_CHUNKS_PER_TILE = 86
EG_PER_TILE = EG_CHUNKS_PER_TILE * EG_CHUNK
EG_VSTRIDE = EG_CHUNK + L
EG_PADDED = NC * NS * EG_PER_TILE
EG_TRIPLES = (EG_CHUNKS_PER_TILE - 2) // 3
EG_ROWBLK = 16
EG_NBLK = N // EG_ROWBLK


def _densify_body(rows_hbm, cols_hbm, vals_hbm, out_hbm,
                  r_buf, c_buf, v_buf, idx2d, stage, sem, acc):
    cid = lax.axis_index("c")
    sid = lax.axis_index("s")

    def _z(i, _):
        stage[pl.ds(i * L, L)] = jnp.zeros((L,), jnp.float32)
        return 0
    lax.fori_loop(0, DN_STAGE // L, _z, 0)
    for k in range(DN_ZSTRIPE // DN_STAGE):
        pltpu.sync_copy(stage, acc.at[pl.ds(sid * DN_ZSTRIPE + k * DN_STAGE,
                                            DN_STAGE)])
    plsc.subcore_barrier()

    t_base = (cid * NS + sid) * DN_PER_TILE

    def _super(s, _):
        off = t_base + s * DN_SUPER
        pltpu.sync_copy(rows_hbm.at[pl.ds(off, DN_SUPER)], r_buf)
        pltpu.sync_copy(cols_hbm.at[pl.ds(off, DN_SUPER)], c_buf)
        pltpu.sync_copy(vals_hbm.at[pl.ds(off, DN_SUPER)], v_buf)

        def _cmp(j, _):
            rv = r_buf[pl.ds(j * L, L)]
            cv = c_buf[pl.ds(j * L, L)]
            flat = rv * IN_DIM + cv
            idx2d[j // (DN_CHUNK // L), pl.ds((j % (DN_CHUNK // L)) * L, L)] = flat
            return 0
        for j in range(DN_SUPER // L):
            _cmp(j, 0)

        descs = [pltpu.async_copy(v_buf.at[pl.ds(k * DN_CHUNK, DN_CHUNK)],
                                  acc.at[idx2d.at[k]], sem, add=True)
                 for k in range(DN_SUPER // DN_CHUNK)]
        for d in descs:
            d.wait()
        return 0
    lax.fori_loop(0, DN_SUPERS_PER_TILE, _super, 0)
    plsc.subcore_barrier()

    for k in range(DN_ZSTRIPE // DN_STAGE):
        off = sid * DN_ZSTRIPE + k * DN_STAGE
        pltpu.sync_copy(acc.at[pl.ds(off, DN_STAGE)], stage)
        pltpu.sync_copy(stage, out_hbm.at[pl.ds(cid * ACC_WORDS + off, DN_STAGE)])


_densify = functools.partial(
    pl.kernel,
    out_type=jax.ShapeDtypeStruct((NC * N * IN_DIM,), jnp.float32),
    mesh=plsc.VectorSubcoreMesh(core_axis_name="c", subcore_axis_name="s"),
    scratch_types=[
        pltpu.VMEM((DN_SUPER,), jnp.int32),
        pltpu.VMEM((DN_SUPER,), jnp.int32),
        pltpu.VMEM((DN_SUPER,), jnp.float32),
        pltpu.VMEM((DN_SUPER // DN_CHUNK, DN_CHUNK), jnp.int32),
        pltpu.VMEM((DN_STAGE,), jnp.float32),
        pltpu.SemaphoreType.DMA,
        pltpu.VMEM_SHARED((ACC_WORDS,), jnp.float32),
    ],
)(_densify_body)


def _sum2_body(p_ref, o_ref):
    o_ref[...] = p_ref[0] + p_ref[1]


def _sum2(parts):
    return pl.pallas_call(
        _sum2_body,
        grid=(N // _BM,),
        in_specs=[pl.BlockSpec((NC, _BM * IN_DIM), lambda i: (0, i))],
        out_specs=pl.BlockSpec((_BM * IN_DIM,), lambda i: (i,)),
        out_shape=jax.ShapeDtypeStruct((N * IN_DIM,), jnp.float32),
    )(parts.reshape(NC, N * IN_DIM))


def _edge_body(src_hbm, dst_hbm, vals_hbm, xd_hbm, out_hbm,
               s2d, d2d, v_vmem, rows_buf, zrows,
               gsem0, gsem1, gsem2, ssem0, ssem1, ssem2, acc):
    cid = lax.axis_index("c")
    sid = lax.axis_index("s")
    wid = sid * NC + cid

    def _z(r, _):
        for j in range(IN_DIM // L):
            zrows[r, pl.ds(j * L, L)] = jnp.zeros((L,), jnp.float32)
        return 0
    lax.fori_loop(0, EG_ROWBLK, _z, 0)

    def _zero_blk(k, _):
        blk = k * NS + sid

        @pl.when(blk < EG_NBLK)
        def _():
            pltpu.sync_copy(zrows, acc.at[pl.ds(blk * EG_ROWBLK, EG_ROWBLK)])
        return 0
    lax.fori_loop(0, (EG_NBLK + NS - 1) // NS, _zero_blk, 0)
    plsc.subcore_barrier()

    e_base = wid * EG_PER_TILE
    g_sems = (gsem0, gsem1, gsem2)
    s_sems = (ssem0, ssem1, ssem2)

    def _loads(c, m):
        off = e_base + c * EG_CHUNK
        pltpu.sync_copy(src_hbm.at[pl.ds(off, EG_CHUNK)], s2d.at[m])
        pltpu.sync_copy(dst_hbm.at[pl.ds(off, EG_CHUNK)], d2d.at[m])
        pltpu.sync_copy(vals_hbm.at[pl.ds(off, EG_CHUNK)],
                        v_vmem.at[pl.ds(m * EG_VSTRIDE, EG_CHUNK)])

    def _issue_gather(m):
        pltpu.async_copy(xd_hbm.at[s2d.at[m]], rows_buf.at[m], g_sems[m])

    def _drain_scatter(m):
        pltpu.make_async_copy(rows_buf.at[m], acc.at[d2d.at[m]],
                              s_sems[m]).wait()

    def _consume(m):
        pltpu.make_async_copy(xd_hbm.at[s2d.at[m]], rows_buf.at[m],
                              g_sems[m]).wait()

        def _scale(i, _):
            val = jnp.full((L,), v_vmem[pl.ds(m * EG_VSTRIDE + i, L)][0],
                           jnp.float32)
            for j in range(IN_DIM // L):
                rows_buf[m, i, pl.ds(j * L, L)] = (
                    rows_buf[m, i, pl.ds(j * L, L)] * val)
            return 0
        lax.fori_loop(0, EG_CHUNK, _scale, 0)
        pltpu.async_copy(rows_buf.at[m], acc.at[d2d.at[m]], s_sems[m],
                         add=True)

    _loads(0, 0)
    _issue_gather(0)

    def _step(c, k, m, m1):

        @pl.when(k >= 2)
        def _():
            _drain_scatter(m1)
        _loads(c + 1, m1)
        _issue_gather(m1)
        _consume(m)

    def _triple(k, _):
        c0 = k * 3
        _step(c0, c0, 0, 1)
        _step(c0 + 1, c0 + 1, 1, 2)
        _step(c0 + 2, c0 + 2, 2, 0)
        return 0
    lax.fori_loop(0, EG_TRIPLES, _triple, 0)

    c0 = EG_TRIPLES * 3
    _drain_scatter(1)
    _loads(c0 + 1, 1)
    _issue_gather(1)
    _consume(0)
    _drain_scatter(2)
    _consume(1)
    _drain_scatter(0)
    _drain_scatter(1)
    plsc.subcore_barrier()

    def _wb_blk(k, _):
        blk = k * NS + sid

        @pl.when(blk < EG_NBLK)
        def _():
            r0 = blk * EG_ROWBLK
            pltpu.sync_copy(acc.at[pl.ds(r0, EG_ROWBLK)], zrows)
            pltpu.sync_copy(zrows, out_hbm.at[cid, pl.ds(r0, EG_ROWBLK)])
        return 0
    lax.fori_loop(0, (EG_NBLK + NS - 1) // NS, _wb_blk, 0)


_edge_agg = functools.partial(
    pl.kernel,
    out_type=jax.ShapeDtypeStruct((NC, N, OUT_DIM), jnp.float32),
    mesh=plsc.VectorSubcoreMesh(core_axis_name="c", subcore_axis_name="s"),
    scratch_types=[
        pltpu.VMEM((3, EG_CHUNK), jnp.int32),
        pltpu.VMEM((3, EG_CHUNK), jnp.int32),
        pltpu.VMEM((3 * EG_VSTRIDE,), jnp.float32),
        pltpu.VMEM((3, EG_CHUNK, IN_DIM), jnp.float32),
        pltpu.VMEM((EG_ROWBLK, IN_DIM), jnp.float32),
        pltpu.SemaphoreType.DMA,
        pltpu.SemaphoreType.DMA,
        pltpu.SemaphoreType.DMA,
        pltpu.SemaphoreType.DMA,
        pltpu.SemaphoreType.DMA,
        pltpu.SemaphoreType.DMA,
        pltpu.VMEM_SHARED((N, IN_DIM), jnp.float32),
    ],
)(_edge_body)


def _matmul_body(p_ref, w_ref, o_ref):
    x = p_ref[0] + p_ref[1]
    y = jnp.dot(x, w_ref[...], preferred_element_type=jnp.float32)
    o_ref[...] = jnp.maximum(y, 0.0)


_BM = 1000


def _matmul_relu(parts, W):
    return pl.pallas_call(
        _matmul_body,
        grid=(N // _BM,),
        in_specs=[
            pl.BlockSpec((NC, _BM, IN_DIM), lambda i: (0, i, 0)),
            pl.BlockSpec((IN_DIM, OUT_DIM), lambda i: (0, 0)),
        ],
        out_specs=pl.BlockSpec((_BM, OUT_DIM), lambda i: (i, 0)),
        out_shape=jax.ShapeDtypeStruct((N, OUT_DIM), jnp.float32),
    )(parts, W)


def kernel(x_rows, x_cols, x_vals, edge_index, adj_vals, W):
    dpad = DN_PADDED - NNZ_X
    dfill = jnp.arange(dpad, dtype=jnp.int32)
    xr = jnp.concatenate([x_rows.astype(jnp.int32), dfill % N])
    xc = jnp.concatenate([x_cols.astype(jnp.int32), dfill % IN_DIM])
    xv = jnp.pad(x_vals, (0, dpad))

    epad = EG_PADDED - E
    efill = jnp.arange(epad, dtype=jnp.int32) % N
    src = jnp.concatenate([edge_index[1].astype(jnp.int32), efill])
    dst = jnp.concatenate([edge_index[0].astype(jnp.int32), efill])
    av = jnp.pad(adj_vals, (0, epad))

    xd = _sum2(_densify(xr, xc, xv)).reshape(N, IN_DIM)
    parts = _edge_agg(src, dst, av, xd)
    return _matmul_relu(parts, W)


# speedup vs baseline: 1.9699x; 1.0986x over previous
import functools

import jax
import jax.numpy as jnp
from jax import lax
from jax.experimental import pallas as pl
from jax.experimental.pallas import tpu as pltpu
from jax.experimental.pallas import tpu_sc as plsc

N = 10000
E = 320000
NNZ_X = 500000
IN_DIM = 128
OUT_DIM = 128

NC = 2
NS = 16
L = 16

DN_CHUNK = 128
DN_SUPER = 2048
DN_SUPERS_PER_TILE = 8
DN_PER_TILE = DN_SUPERS_PER_TILE * DN_SUPER
DN_PADDED = NC * NS * DN_PER_TILE
ACC_WORDS = N * IN_DIM
DN_ZSTRIPE = ACC_WORDS // NS
DN_STAGE = 8000

EG_CHUNK = 120
EG_CHUNKS_PER_TILE = 86
EG_PER_TILE = EG_CHUNKS_PER_TILE * EG_CHUNK
EG_VSTRIDE = EG_CHUNK + L
EG_PADDED = NC * NS * EG_PER_TILE
EG_TRIPLES = (EG_CHUNKS_PER_TILE - 2) // 3
EG_ROWBLK = 16
EG_NBLK = N // EG_ROWBLK


def _densify_body(rows_hbm, cols_hbm, vals_hbm, out_hbm,
                  r_buf, c_buf, v_buf, idx2d, stage, sem, acc):
    cid = lax.axis_index("c")
    sid = lax.axis_index("s")

    def _z(i, _):
        stage[pl.ds(i * L, L)] = jnp.zeros((L,), jnp.float32)
        return 0
    lax.fori_loop(0, DN_STAGE // L, _z, 0)
    for k in range(DN_ZSTRIPE // DN_STAGE):
        pltpu.sync_copy(stage, acc.at[pl.ds(sid * DN_ZSTRIPE + k * DN_STAGE,
                                            DN_STAGE)])
    plsc.subcore_barrier()

    t_base = (cid * NS + sid) * DN_PER_TILE

    def _super(s, _):
        off = t_base + s * DN_SUPER
        pltpu.sync_copy(rows_hbm.at[pl.ds(off, DN_SUPER)], r_buf)
        pltpu.sync_copy(cols_hbm.at[pl.ds(off, DN_SUPER)], c_buf)
        pltpu.sync_copy(vals_hbm.at[pl.ds(off, DN_SUPER)], v_buf)

        def _cmp(j, _):
            rv = r_buf[pl.ds(j * L, L)]
            cv = c_buf[pl.ds(j * L, L)]
            flat = rv * IN_DIM + cv
            idx2d[j // (DN_CHUNK // L), pl.ds((j % (DN_CHUNK // L)) * L, L)] = flat
            return 0
        for j in range(DN_SUPER // L):
            _cmp(j, 0)

        descs = [pltpu.async_copy(v_buf.at[pl.ds(k * DN_CHUNK, DN_CHUNK)],
                                  acc.at[idx2d.at[k]], sem, add=True)
                 for k in range(DN_SUPER // DN_CHUNK)]
        for d in descs:
            d.wait()
        return 0
    lax.fori_loop(0, DN_SUPERS_PER_TILE, _super, 0)
    plsc.subcore_barrier()

    for k in range(DN_ZSTRIPE // DN_STAGE):
        off = sid * DN_ZSTRIPE + k * DN_STAGE
        pltpu.sync_copy(acc.at[pl.ds(off, DN_STAGE)], stage)
        pltpu.sync_copy(stage, out_hbm.at[pl.ds(cid * ACC_WORDS + off, DN_STAGE)])


_densify = functools.partial(
    pl.kernel,
    out_type=jax.ShapeDtypeStruct((NC * N * IN_DIM,), jnp.float32),
    mesh=plsc.VectorSubcoreMesh(core_axis_name="c", subcore_axis_name="s"),
    scratch_types=[
        pltpu.VMEM((DN_SUPER,), jnp.int32),
        pltpu.VMEM((DN_SUPER,), jnp.int32),
        pltpu.VMEM((DN_SUPER,), jnp.float32),
        pltpu.VMEM((DN_SUPER // DN_CHUNK, DN_CHUNK), jnp.int32),
        pltpu.VMEM((DN_STAGE,), jnp.float32),
        pltpu.SemaphoreType.DMA,
        pltpu.VMEM_SHARED((ACC_WORDS,), jnp.float32),
    ],
)(_densify_body)


def _sum2_body(p_ref, o_ref):
    o_ref[...] = p_ref[0] + p_ref[1]


def _sum2(parts):
    return pl.pallas_call(
        _sum2_body,
        grid=(N // _BM,),
        in_specs=[pl.BlockSpec((NC, _BM * IN_DIM), lambda i: (0, i))],
        out_specs=pl.BlockSpec((_BM * IN_DIM,), lambda i: (i,)),
        out_shape=jax.ShapeDtypeStruct((N * IN_DIM,), jnp.float32),
    )(parts.reshape(NC, N * IN_DIM))


def _edge_body(src_hbm, dst_hbm, vals_hbm, xd_hbm, out_hbm,
               s2d, d2d, v_vmem, rows_buf, zrows,
               gsem0, gsem1, gsem2, ssem0, ssem1, ssem2, acc):
    cid = lax.axis_index("c")
    sid = lax.axis_index("s")
    wid = sid * NC + cid

    def _z(r, _):
        for j in range(IN_DIM // L):
            zrows[r, pl.ds(j * L, L)] = jnp.zeros((L,), jnp.float32)
        return 0
    lax.fori_loop(0, EG_ROWBLK, _z, 0)

    def _zero_blk(k, _):
        blk = k * NS + sid

        @pl.when(blk < EG_NBLK)
        def _():
            pltpu.sync_copy(zrows, acc.at[pl.ds(blk * EG_ROWBLK, EG_ROWBLK)])
        return 0
    lax.fori_loop(0, (EG_NBLK + NS - 1) // NS, _zero_blk, 0)
    plsc.subcore_barrier()

    e_base = wid * EG_PER_TILE
    g_sems = (gsem0, gsem1, gsem2)
    s_sems = (ssem0, ssem1, ssem2)

    def _loads(c, m):
        off = e_base + c * EG_CHUNK
        pltpu.sync_copy(src_hbm.at[pl.ds(off, EG_CHUNK)], s2d.at[m])
        pltpu.sync_copy(dst_hbm.at[pl.ds(off, EG_CHUNK)], d2d.at[m])
        pltpu.sync_copy(vals_hbm.at[pl.ds(off, EG_CHUNK)],
                        v_vmem.at[pl.ds(m * EG_VSTRIDE, EG_CHUNK)])

    def _issue_gather(m):
        pltpu.async_copy(xd_hbm.at[s2d.at[m]], rows_buf.at[m], g_sems[m])

    def _drain_scatter(m):
        pltpu.make_async_copy(rows_buf.at[m], acc.at[d2d.at[m]],
                              s_sems[m]).wait()

    def _consume(m):
        pltpu.make_async_copy(xd_hbm.at[s2d.at[m]], rows_buf.at[m],
                              g_sems[m]).wait()

        def _scale(i2, _):
            i = i2 * 2
            v0 = jnp.full((L,), v_vmem[pl.ds(m * EG_VSTRIDE + i, L)][0],
                          jnp.float32)
            v1 = jnp.full((L,), v_vmem[pl.ds(m * EG_VSTRIDE + i + 1, L)][0],
                          jnp.float32)
            for j in range(IN_DIM // L):
                rows_buf[m, i, pl.ds(j * L, L)] = (
                    rows_buf[m, i, pl.ds(j * L, L)] * v0)
                rows_buf[m, i + 1, pl.ds(j * L, L)] = (
                    rows_buf[m, i + 1, pl.ds(j * L, L)] * v1)
            return 0
        lax.fori_loop(0, EG_CHUNK // 2, _scale, 0)
        pltpu.async_copy(rows_buf.at[m], acc.at[d2d.at[m]], s_sems[m],
                         add=True)

    _loads(0, 0)
    _issue_gather(0)

    def _step(c, k, m, m1):

        @pl.when(k >= 2)
        def _():
            _drain_scatter(m1)
        _loads(c + 1, m1)
        _issue_gather(m1)
        _consume(m)

    def _triple(k, _):
        c0 = k * 3
        _step(c0, c0, 0, 1)
        _step(c0 + 1, c0 + 1, 1, 2)
        _step(c0 + 2, c0 + 2, 2, 0)
        return 0
    lax.fori_loop(0, EG_TRIPLES, _triple, 0)

    c0 = EG_TRIPLES * 3
    _drain_scatter(1)
    _loads(c0 + 1, 1)
    _issue_gather(1)
    _consume(0)
    _drain_scatter(2)
    _consume(1)
    _drain_scatter(0)
    _drain_scatter(1)
    plsc.subcore_barrier()

    def _wb_blk(k, _):
        blk = k * NS + sid

        @pl.when(blk < EG_NBLK)
        def _():
            r0 = blk * EG_ROWBLK
            pltpu.sync_copy(acc.at[pl.ds(r0, EG_ROWBLK)], zrows)
            pltpu.sync_copy(zrows, out_hbm.at[cid, pl.ds(r0, EG_ROWBLK)])
        return 0
    lax.fori_loop(0, (EG_NBLK + NS - 1) // NS, _wb_blk, 0)


_edge_agg = functools.partial(
    pl.kernel,
    out_type=jax.ShapeDtypeStruct((NC, N, OUT_DIM), jnp.float32),
    mesh=plsc.VectorSubcoreMesh(core_axis_name="c", subcore_axis_name="s"),
    scratch_types=[
        pltpu.VMEM((3, EG_CHUNK), jnp.int32),
        pltpu.VMEM((3, EG_CHUNK), jnp.int32),
        pltpu.VMEM((3 * EG_VSTRIDE,), jnp.float32),
        pltpu.VMEM((3, EG_CHUNK, IN_DIM), jnp.float32),
        pltpu.VMEM((EG_ROWBLK, IN_DIM), jnp.float32),
        pltpu.SemaphoreType.DMA,
        pltpu.SemaphoreType.DMA,
        pltpu.SemaphoreType.DMA,
        pltpu.SemaphoreType.DMA,
        pltpu.SemaphoreType.DMA,
        pltpu.SemaphoreType.DMA,
        pltpu.VMEM_SHARED((N, IN_DIM), jnp.float32),
    ],
)(_edge_body)


def _matmul_body(p_ref, w_ref, o_ref):
    x = p_ref[0] + p_ref[1]
    y = jnp.dot(x, w_ref[...], preferred_element_type=jnp.float32)
    o_ref[...] = jnp.maximum(y, 0.0)


_BM = 1000


def _matmul_relu(parts, W):
    return pl.pallas_call(
        _matmul_body,
        grid=(N // _BM,),
        in_specs=[
            pl.BlockSpec((NC, _BM, IN_DIM), lambda i: (0, i, 0)),
            pl.BlockSpec((IN_DIM, OUT_DIM), lambda i: (0, 0)),
        ],
        out_specs=pl.BlockSpec((_BM, OUT_DIM), lambda i: (i, 0)),
        out_shape=jax.ShapeDtypeStruct((N, OUT_DIM), jnp.float32),
    )(parts, W)


def kernel(x_rows, x_cols, x_vals, edge_index, adj_vals, W):
    dpad = DN_PADDED - NNZ_X
    dfill = jnp.arange(dpad, dtype=jnp.int32)
    xr = jnp.concatenate([x_rows.astype(jnp.int32), dfill % N])
    xc = jnp.concatenate([x_cols.astype(jnp.int32), dfill % IN_DIM])
    xv = jnp.pad(x_vals, (0, dpad))

    epad = EG_PADDED - E
    efill = jnp.arange(epad, dtype=jnp.int32) % N
    src = jnp.concatenate([edge_index[1].astype(jnp.int32), efill])
    dst = jnp.concatenate([edge_index[0].astype(jnp.int32), efill])
    av = jnp.pad(adj_vals, (0, epad))

    xd = _sum2(_densify(xr, xc, xv)).reshape(N, IN_DIM)
    parts = _edge_agg(src, dst, av, xd)
    return _matmul_relu(parts, W)


# speedup vs baseline: 2.0304x; 1.0307x over previous
import functools

import jax
import jax.numpy as jnp
from jax import lax
from jax.experimental import pallas as pl
from jax.experimental.pallas import tpu as pltpu
from jax.experimental.pallas import tpu_sc as plsc

N = 10000
E = 320000
NNZ_X = 500000
IN_DIM = 128
OUT_DIM = 128

NC = 2
NS = 16
L = 16

DN_CHUNK = 128
DN_SUPER = 2048
DN_SUPERS_PER_TILE = 8
DN_PER_TILE = DN_SUPERS_PER_TILE * DN_SUPER
DN_PADDED = NC * NS * DN_PER_TILE
ACC_WORDS = N * IN_DIM
DN_ZSTRIPE = ACC_WORDS // NS
DN_STAGE = 8000

EG_CHUNK = 120
EG_CHUNKS_PER_TILE = 86
EG_PER_TILE = EG_CHUNKS_PER_TILE * EG_CHUNK
EG_VSTRIDE = EG_CHUNK + L
EG_PADDED = NC * NS * EG_PER_TILE
EG_TRIPLES = (EG_CHUNKS_PER_TILE - 2) // 3
EG_ROWBLK = 16
EG_NBLK = N // EG_ROWBLK


def _densify_body(rows_hbm, cols_hbm, vals_hbm, out_hbm,
                  r_buf, c_buf, v_buf, idx2d, stage, sem, acc):
    cid = lax.axis_index("c")
    sid = lax.axis_index("s")

    def _z(i, _):
        stage[pl.ds(i * L, L)] = jnp.zeros((L,), jnp.float32)
        return 0
    lax.fori_loop(0, DN_STAGE // L, _z, 0)
    for k in range(DN_ZSTRIPE // DN_STAGE):
        pltpu.sync_copy(stage, acc.at[pl.ds(sid * DN_ZSTRIPE + k * DN_STAGE,
                                            DN_STAGE)])
    plsc.subcore_barrier()

    t_base = (cid * NS + sid) * DN_PER_TILE

    def _super(s, _):
        off = t_base + s * DN_SUPER
        pltpu.sync_copy(rows_hbm.at[pl.ds(off, DN_SUPER)], r_buf)
        pltpu.sync_copy(cols_hbm.at[pl.ds(off, DN_SUPER)], c_buf)
        pltpu.sync_copy(vals_hbm.at[pl.ds(off, DN_SUPER)], v_buf)

        def _cmp(j, _):
            rv = r_buf[pl.ds(j * L, L)]
            cv = c_buf[pl.ds(j * L, L)]
            flat = rv * IN_DIM + cv
            idx2d[j // (DN_CHUNK // L), pl.ds((j % (DN_CHUNK // L)) * L, L)] = flat
            return 0
        for j in range(DN_SUPER // L):
            _cmp(j, 0)

        descs = [pltpu.async_copy(v_buf.at[pl.ds(k * DN_CHUNK, DN_CHUNK)],
                                  acc.at[idx2d.at[k]], sem, add=True)
                 for k in range(DN_SUPER // DN_CHUNK)]
        for d in descs:
            d.wait()
        return 0
    lax.fori_loop(0, DN_SUPERS_PER_TILE, _super, 0)
    plsc.subcore_barrier()

    for k in range(DN_ZSTRIPE // DN_STAGE):
        off = sid * DN_ZSTRIPE + k * DN_STAGE
        pltpu.sync_copy(acc.at[pl.ds(off, DN_STAGE)], stage)
        pltpu.sync_copy(stage, out_hbm.at[pl.ds(cid * ACC_WORDS + off, DN_STAGE)])


_densify = functools.partial(
    pl.kernel,
    out_type=jax.ShapeDtypeStruct((NC * N * IN_DIM,), jnp.float32),
    mesh=plsc.VectorSubcoreMesh(core_axis_name="c", subcore_axis_name="s"),
    scratch_types=[
        pltpu.VMEM((DN_SUPER,), jnp.int32),
        pltpu.VMEM((DN_SUPER,), jnp.int32),
        pltpu.VMEM((DN_SUPER,), jnp.float32),
        pltpu.VMEM((DN_SUPER // DN_CHUNK, DN_CHUNK), jnp.int32),
        pltpu.VMEM((DN_STAGE,), jnp.float32),
        pltpu.SemaphoreType.DMA,
        pltpu.VMEM_SHARED((ACC_WORDS,), jnp.float32),
    ],
)(_densify_body)


def _sum2_body(p_ref, o_ref):
    o_ref[...] = p_ref[0] + p_ref[1]


def _sum2(parts):
    return pl.pallas_call(
        _sum2_body,
        grid=(N // _BM,),
        in_specs=[pl.BlockSpec((NC, _BM * IN_DIM), lambda i: (0, i))],
        out_specs=pl.BlockSpec((_BM * IN_DIM,), lambda i: (i,)),
        out_shape=jax.ShapeDtypeStruct((N * IN_DIM,), jnp.float32),
    )(parts.reshape(NC, N * IN_DIM))


def _edge_body(src_hbm, dst_hbm, vals_hbm, xd_hbm, out_hbm,
               s2d, d2d, v_vmem, rows_buf, zrows,
               gsem0, gsem1, gsem2, ssem0, ssem1, ssem2, acc):
    cid = lax.axis_index("c")
    sid = lax.axis_index("s")
    wid = sid * NC + cid

    def _z(r, _):
        for j in range(IN_DIM // L):
            zrows[r, pl.ds(j * L, L)] = jnp.zeros((L,), jnp.float32)
        return 0
    lax.fori_loop(0, EG_ROWBLK, _z, 0)

    def _zero_blk(k, _):
        blk = k * NS + sid

        @pl.when(blk < EG_NBLK)
        def _():
            pltpu.sync_copy(zrows, acc.at[pl.ds(blk * EG_ROWBLK, EG_ROWBLK)])
        return 0
    lax.fori_loop(0, (EG_NBLK + NS - 1) // NS, _zero_blk, 0)
    plsc.subcore_barrier()

    e_base = wid * EG_PER_TILE
    g_sems = (gsem0, gsem1, gsem2)
    s_sems = (ssem0, ssem1, ssem2)

    def _loads(c, m):
        off = e_base + c * EG_CHUNK
        pltpu.sync_copy(src_hbm.at[pl.ds(off, EG_CHUNK)], s2d.at[m])
        pltpu.sync_copy(dst_hbm.at[pl.ds(off, EG_CHUNK)], d2d.at[m])
        pltpu.sync_copy(vals_hbm.at[pl.ds(off, EG_CHUNK)],
                        v_vmem.at[pl.ds(m * EG_VSTRIDE, EG_CHUNK)])

    def _issue_gather(m):
        pltpu.async_copy(xd_hbm.at[s2d.at[m]], rows_buf.at[m], g_sems[m])

    def _drain_scatter(m):
        pltpu.make_async_copy(rows_buf.at[m], acc.at[d2d.at[m]],
                              s_sems[m]).wait()

    def _consume(m):
        pltpu.make_async_copy(xd_hbm.at[s2d.at[m]], rows_buf.at[m],
                              g_sems[m]).wait()

        def _scale(i4, _):
            i = i4 * 4
            vs = [jnp.full((L,),
                           v_vmem[pl.ds(m * EG_VSTRIDE + i + u, L)][0],
                           jnp.float32) for u in range(4)]
            for j in range(IN_DIM // L):
                for u in range(4):
                    rows_buf[m, i + u, pl.ds(j * L, L)] = (
                        rows_buf[m, i + u, pl.ds(j * L, L)] * vs[u])
            return 0
        lax.fori_loop(0, EG_CHUNK // 4, _scale, 0)
        pltpu.async_copy(rows_buf.at[m], acc.at[d2d.at[m]], s_sems[m],
                         add=True)

    _loads(0, 0)
    _issue_gather(0)

    def _step(c, k, m, m1):

        @pl.when(k >= 2)
        def _():
            _drain_scatter(m1)
        _loads(c + 1, m1)
        _issue_gather(m1)
        _consume(m)

    def _triple(k, _):
        c0 = k * 3
        _step(c0, c0, 0, 1)
        _step(c0 + 1, c0 + 1, 1, 2)
        _step(c0 + 2, c0 + 2, 2, 0)
        return 0
    lax.fori_loop(0, EG_TRIPLES, _triple, 0)

    c0 = EG_TRIPLES * 3
    _drain_scatter(1)
    _loads(c0 + 1, 1)
    _issue_gather(1)
    _consume(0)
    _drain_scatter(2)
    _consume(1)
    _drain_scatter(0)
    _drain_scatter(1)
    plsc.subcore_barrier()

    def _wb_blk(k, _):
        blk = k * NS + sid

        @pl.when(blk < EG_NBLK)
        def _():
            r0 = blk * EG_ROWBLK
            pltpu.sync_copy(acc.at[pl.ds(r0, EG_ROWBLK)], zrows)
            pltpu.sync_copy(zrows, out_hbm.at[cid, pl.ds(r0, EG_ROWBLK)])
        return 0
    lax.fori_loop(0, (EG_NBLK + NS - 1) // NS, _wb_blk, 0)


_edge_agg = functools.partial(
    pl.kernel,
    out_type=jax.ShapeDtypeStruct((NC, N, OUT_DIM), jnp.float32),
    mesh=plsc.VectorSubcoreMesh(core_axis_name="c", subcore_axis_name="s"),
    scratch_types=[
        pltpu.VMEM((3, EG_CHUNK), jnp.int32),
        pltpu.VMEM((3, EG_CHUNK), jnp.int32),
        pltpu.VMEM((3 * EG_VSTRIDE,), jnp.float32),
        pltpu.VMEM((3, EG_CHUNK, IN_DIM), jnp.float32),
        pltpu.VMEM((EG_ROWBLK, IN_DIM), jnp.float32),
        pltpu.SemaphoreType.DMA,
        pltpu.SemaphoreType.DMA,
        pltpu.SemaphoreType.DMA,
        pltpu.SemaphoreType.DMA,
        pltpu.SemaphoreType.DMA,
        pltpu.SemaphoreType.DMA,
        pltpu.VMEM_SHARED((N, IN_DIM), jnp.float32),
    ],
)(_edge_body)


def _matmul_body(p_ref, w_ref, o_ref):
    x = p_ref[0] + p_ref[1]
    y = jnp.dot(x, w_ref[...], preferred_element_type=jnp.float32)
    o_ref[...] = jnp.maximum(y, 0.0)


_BM = 1000


def _matmul_relu(parts, W):
    return pl.pallas_call(
        _matmul_body,
        grid=(N // _BM,),
        in_specs=[
            pl.BlockSpec((NC, _BM, IN_DIM), lambda i: (0, i, 0)),
            pl.BlockSpec((IN_DIM, OUT_DIM), lambda i: (0, 0)),
        ],
        out_specs=pl.BlockSpec((_BM, OUT_DIM), lambda i: (i, 0)),
        out_shape=jax.ShapeDtypeStruct((N, OUT_DIM), jnp.float32),
    )(parts, W)


def kernel(x_rows, x_cols, x_vals, edge_index, adj_vals, W):
    dpad = DN_PADDED - NNZ_X
    dfill = jnp.arange(dpad, dtype=jnp.int32)
    xr = jnp.concatenate([x_rows.astype(jnp.int32), dfill % N])
    xc = jnp.concatenate([x_cols.astype(jnp.int32), dfill % IN_DIM])
    xv = jnp.pad(x_vals, (0, dpad))

    epad = EG_PADDED - E
    efill = jnp.arange(epad, dtype=jnp.int32) % N
    src = jnp.concatenate([edge_index[1].astype(jnp.int32), efill])
    dst = jnp.concatenate([edge_index[0].astype(jnp.int32), efill])
    av = jnp.pad(adj_vals, (0, epad))

    xd = _sum2(_densify(xr, xc, xv)).reshape(N, IN_DIM)
    parts = _edge_agg(src, dst, av, xd)
    return _matmul_relu(parts, W)
